# Initial kernel scaffold; baseline (speedup 1.0000x reference)
import jax
import jax.numpy as jnp
from jax.experimental import pallas as pl


def kernel(inputs_nchw, targets):
    raise NotImplementedError("write your pallas kernel here")


# speedup vs baseline: 1.0819x; 1.0819x over previous
import functools

import jax
import jax.numpy as jnp
from jax import lax
from jax.experimental import pallas as pl
from jax.experimental.pallas import tpu as pltpu

_EPS = 1e-12
_K = 20


def _round_up(a, b):
    return pl.cdiv(a, b) * b


def _stats_kernel(x_ref, t_ref, sums_ref, nsums_ref, counts_ref, *,
                  num_classes, ragged):
    K = num_classes

    @pl.when(pl.program_id(1) == 0)
    def _():
        sums_ref[...] = jnp.zeros_like(sums_ref)
        nsums_ref[...] = jnp.zeros_like(nsums_ref)
        counts_ref[...] = jnp.zeros_like(counts_ref)

    x = x_ref[...]
    t = t_ref[...]

    if ragged:
        x = jnp.where(t < K, x, jnp.zeros_like(x))

    xf = x.astype(jnp.float32)
    C, T = xf.shape

    xsq = xf * xf
    ones_row = jnp.ones((1, C), jnp.float32)
    norm2 = lax.dot_general(ones_row, xsq, (((1,), (0,)), ((), ())),
                            preferred_element_type=jnp.float32)
    invnorm = lax.rsqrt(norm2 + _EPS)

    class_ids = lax.broadcasted_iota(jnp.int32, (K, T), 0)
    mask = class_ids == t
    onehot = jnp.where(mask, 1.0, 0.0).astype(jnp.float32)
    scaled = jnp.where(mask, jnp.broadcast_to(invnorm, (K, T)),
                       0.0).astype(jnp.float32)

    dims = (((1,), (1,)), ((), ()))
    sums_ref[...] += lax.dot_general(onehot, xf, dims,
                                     preferred_element_type=jnp.float32)
    nsums_ref[...] += lax.dot_general(scaled, xf, dims,
                                      preferred_element_type=jnp.float32)
    counts_ref[...] += jnp.sum(onehot, axis=1, keepdims=True)


def _loss_kernel(sums_ref, nsums_ref, counts_ref, out_ref, *, num_classes):
    K = num_classes
    sums = jnp.sum(sums_ref[...], axis=0)
    nsums = jnp.sum(nsums_ref[...], axis=0)
    counts = jnp.sum(counts_ref[...], axis=0)

    valid = counts > 0.0
    sum_pixel = jnp.maximum(counts, 1.0)
    centers = sums / sum_pixel

    cn2 = jnp.sum(centers * centers, axis=1, keepdims=True) + _EPS
    norms = jnp.sqrt(cn2)
    rn = 1.0 / norms

    dot_nc = jnp.sum(nsums * centers, axis=1, keepdims=True)
    sim_per = 1.0 - dot_nc / (norms * sum_pixel)
    sim_loss = jnp.sum(jnp.where(valid, sim_per, 0.0), keepdims=True)

    gram = lax.dot_general(centers, centers, (((1,), (1,)), ((), ())),
                           preferred_element_type=jnp.float32)
    ids_r = lax.broadcasted_iota(jnp.int32, (K, K), 0)
    ids_c = lax.broadcasted_iota(jnp.int32, (K, K), 1)
    offdiag = jnp.where(ids_r == ids_c, 0.0, jnp.maximum(gram, 0.0))
    colsum = lax.dot_general(offdiag, rn, (((1,), (0,)), ((), ())),
                             preferred_element_type=jnp.float32)
    diag_cos = (cn2 - _EPS) / cn2
    per_row = (colsum * rn + (1.0 - diag_cos)) / K
    diff_loss = jnp.sum(jnp.where(valid, per_row, 0.0), keepdims=True)

    out_ref[...] = sim_loss + diff_loss


def _embedding_loss(inputs_nchw, targets, num_classes, *, tile_hw=16384):
    N, C, H, W = inputs_nchw.shape
    HW = H * W
    K = num_classes

    x = inputs_nchw.reshape(N, C, HW)
    t = targets.reshape(N, 1, HW).astype(jnp.int32)

    n_tiles = max(1, pl.cdiv(HW, tile_hw))
    if n_tiles > 1 and n_tiles % 2 == 1:
        n_tiles += 1
    tile = _round_up(pl.cdiv(HW, n_tiles), 128)
    n_tiles = pl.cdiv(HW, tile)
    n_split = 2 if (n_tiles >= 2 and n_tiles % 2 == 0) else 1
    tps = n_tiles // n_split

    HW_pad = n_tiles * tile
    ragged = HW_pad != HW
    if ragged:
        t = jnp.pad(t, ((0, 0), (0, 0), (0, HW_pad - HW)),
                    constant_values=num_classes)

    grid = (n_split, N * tps)

    sums_n, nsums_n, counts_n = pl.pallas_call(
        functools.partial(_stats_kernel, num_classes=K, ragged=ragged),
        out_shape=(jax.ShapeDtypeStruct((n_split, K, C), jnp.float32),
                   jax.ShapeDtypeStruct((n_split, K, C), jnp.float32),
                   jax.ShapeDtypeStruct((n_split, K, 1), jnp.float32)),
        grid_spec=pltpu.PrefetchScalarGridSpec(
            num_scalar_prefetch=0, grid=grid,
            in_specs=[pl.BlockSpec((None, C, tile),
                                   lambda s, q: (q // tps, 0,
                                                 s * tps + q % tps)),
                      pl.BlockSpec((None, 1, tile),
                                   lambda s, q: (q // tps, 0,
                                                 s * tps + q % tps))],
            out_specs=[pl.BlockSpec((None, K, C), lambda s, q: (s, 0, 0)),
                       pl.BlockSpec((None, K, C), lambda s, q: (s, 0, 0)),
                       pl.BlockSpec((None, K, 1), lambda s, q: (s, 0, 0))]),
        compiler_params=pltpu.CompilerParams(
            dimension_semantics=("parallel", "arbitrary")),
    )(x, t)

    loss = pl.pallas_call(
        functools.partial(_loss_kernel, num_classes=K),
        out_shape=jax.ShapeDtypeStruct((1, 1), jnp.float32),
    )(sums_n, nsums_n, counts_n)

    return loss.reshape(1)


def kernel(inputs_nchw, targets):
    return _embedding_loss(inputs_nchw, targets, _K)


# speedup vs baseline: 2.3339x; 2.1571x over previous
import functools

import jax
import jax.numpy as jnp
from jax import lax
from jax.experimental import pallas as pl
from jax.experimental.pallas import tpu as pltpu

_EPS = 1e-12
_K = 20


def _stats_kernel(x_ref, t_ref, sums_ref, nsums_ref, counts_ref, *,
                  num_classes):
    K = num_classes

    @pl.when(pl.program_id(1) == 0)
    def _():
        sums_ref[...] = jnp.zeros_like(sums_ref)
        nsums_ref[...] = jnp.zeros_like(nsums_ref)
        counts_ref[...] = jnp.zeros_like(counts_ref)

    x = x_ref[...]
    t = t_ref[...]

    xf = x.astype(jnp.float32)
    C, R, W = xf.shape
    T = R * W

    xflat = xf.reshape(C, T)
    tflat = t.reshape(1, T)

    xsq = xflat * xflat
    ones_row = jnp.ones((1, C), jnp.float32)
    norm2 = lax.dot_general(ones_row, xsq, (((1,), (0,)), ((), ())),
                            preferred_element_type=jnp.float32)
    invnorm = lax.rsqrt(norm2 + _EPS)

    class_ids = lax.broadcasted_iota(jnp.int32, (K, T), 0)
    mask = class_ids == tflat
    onehot = jnp.where(mask, 1.0, 0.0).astype(jnp.float32)
    scaled = jnp.where(mask, jnp.broadcast_to(invnorm, (K, T)),
                       0.0).astype(jnp.float32)

    dims = (((1,), (1,)), ((), ()))
    sums_ref[...] += lax.dot_general(onehot, xflat, dims,
                                     preferred_element_type=jnp.float32)
    nsums_ref[...] += lax.dot_general(scaled, xflat, dims,
                                      preferred_element_type=jnp.float32)
    counts_ref[...] += jnp.sum(onehot, axis=1, keepdims=True)


def _loss_kernel(sums_ref, nsums_ref, counts_ref, out_ref, *, num_classes):
    K = num_classes
    sums = jnp.sum(sums_ref[...], axis=0)
    nsums = jnp.sum(nsums_ref[...], axis=0)
    counts = jnp.sum(counts_ref[...], axis=0)

    valid = counts > 0.0
    sum_pixel = jnp.maximum(counts, 1.0)
    centers = sums / sum_pixel

    cn2 = jnp.sum(centers * centers, axis=1, keepdims=True) + _EPS
    norms = jnp.sqrt(cn2)
    rn = 1.0 / norms

    dot_nc = jnp.sum(nsums * centers, axis=1, keepdims=True)
    sim_per = 1.0 - dot_nc / (norms * sum_pixel)
    sim_loss = jnp.sum(jnp.where(valid, sim_per, 0.0), keepdims=True)

    gram = lax.dot_general(centers, centers, (((1,), (1,)), ((), ())),
                           preferred_element_type=jnp.float32)
    ids_r = lax.broadcasted_iota(jnp.int32, (K, K), 0)
    ids_c = lax.broadcasted_iota(jnp.int32, (K, K), 1)
    offdiag = jnp.where(ids_r == ids_c, 0.0, jnp.maximum(gram, 0.0))
    colsum = lax.dot_general(offdiag, rn, (((1,), (0,)), ((), ())),
                             preferred_element_type=jnp.float32)
    diag_cos = (cn2 - _EPS) / cn2
    per_row = (colsum * rn + (1.0 - diag_cos)) / K
    diff_loss = jnp.sum(jnp.where(valid, per_row, 0.0), keepdims=True)

    out_ref[...] = sim_loss + diff_loss


def _embedding_loss(inputs_nchw, targets, num_classes, *, tile_rows=64):
    N, C, H, W = inputs_nchw.shape
    K = num_classes
    t = targets.astype(jnp.int32)

    R = tile_rows
    while R > 8 and H % (2 * R) != 0:
        R //= 2
    if H % R != 0:
        R = H
    n_tiles = H // R
    n_split = 2 if (n_tiles >= 2 and n_tiles % 2 == 0) else 1
    tps = n_tiles // n_split

    grid = (n_split, N * tps)

    sums_n, nsums_n, counts_n = pl.pallas_call(
        functools.partial(_stats_kernel, num_classes=K),
        out_shape=(jax.ShapeDtypeStruct((n_split, K, C), jnp.float32),
                   jax.ShapeDtypeStruct((n_split, K, C), jnp.float32),
                   jax.ShapeDtypeStruct((n_split, K, 1), jnp.float32)),
        grid_spec=pltpu.PrefetchScalarGridSpec(
            num_scalar_prefetch=0, grid=grid,
            in_specs=[pl.BlockSpec((None, C, R, W),
                                   lambda s, q: (q // tps, 0,
                                                 s * tps + q % tps, 0)),
                      pl.BlockSpec((None, R, W),
                                   lambda s, q: (q // tps,
                                                 s * tps + q % tps, 0))],
            out_specs=[pl.BlockSpec((None, K, C), lambda s, q: (s, 0, 0)),
                       pl.BlockSpec((None, K, C), lambda s, q: (s, 0, 0)),
                       pl.BlockSpec((None, K, 1), lambda s, q: (s, 0, 0))]),
        compiler_params=pltpu.CompilerParams(
            dimension_semantics=("parallel", "arbitrary")),
    )(inputs_nchw, t)

    loss = pl.pallas_call(
        functools.partial(_loss_kernel, num_classes=K),
        out_shape=jax.ShapeDtypeStruct((1, 1), jnp.float32),
    )(sums_n, nsums_n, counts_n)

    return loss.reshape(1)


def kernel(inputs_nchw, targets):
    return _embedding_loss(inputs_nchw, targets, _K)


# speedup vs baseline: 2.4409x; 1.0459x over previous
import functools

import jax
import jax.numpy as jnp
from jax import lax
from jax.experimental import pallas as pl
from jax.experimental.pallas import tpu as pltpu

_EPS = 1e-12
_K = 20


def _stats_loss_kernel(x_ref, t_ref, out_ref, sums_ref, nsums_ref,
                       counts_ref, *, num_classes, n_steps):
    K = num_classes
    q = pl.program_id(0)

    @pl.when(q == 0)
    def _():
        sums_ref[...] = jnp.zeros_like(sums_ref)
        nsums_ref[...] = jnp.zeros_like(nsums_ref)
        counts_ref[...] = jnp.zeros_like(counts_ref)

    x = x_ref[...]
    t = t_ref[...]

    xf = x.astype(jnp.float32)
    C, R, W = xf.shape
    T = R * W

    xflat = xf.reshape(C, T)
    tflat = t.reshape(1, T)

    xsq = xflat * xflat
    ones_row = jnp.ones((1, C), jnp.float32)
    norm2 = lax.dot_general(ones_row, xsq, (((1,), (0,)), ((), ())),
                            preferred_element_type=jnp.float32)
    invnorm = lax.rsqrt(norm2 + _EPS)

    class_ids = lax.broadcasted_iota(jnp.int32, (K, T), 0)
    mask = class_ids == tflat
    onehot = jnp.where(mask, 1.0, 0.0).astype(jnp.float32)
    scaled = jnp.where(mask, jnp.broadcast_to(invnorm, (K, T)),
                       0.0).astype(jnp.float32)

    dims = (((1,), (1,)), ((), ()))
    sums_ref[...] += lax.dot_general(onehot, xflat, dims,
                                     preferred_element_type=jnp.float32)
    nsums_ref[...] += lax.dot_general(scaled, xflat, dims,
                                      preferred_element_type=jnp.float32)
    counts_ref[...] += jnp.sum(onehot, axis=1, keepdims=True)

    @pl.when(q == n_steps - 1)
    def _():
        sums = sums_ref[...]
        nsums = nsums_ref[...]
        counts = counts_ref[...]

        valid = counts > 0.0
        sum_pixel = jnp.maximum(counts, 1.0)
        centers = sums / sum_pixel

        cn2 = jnp.sum(centers * centers, axis=1, keepdims=True) + _EPS
        norms = jnp.sqrt(cn2)
        rn = 1.0 / norms

        dot_nc = jnp.sum(nsums * centers, axis=1, keepdims=True)
        sim_per = 1.0 - dot_nc / (norms * sum_pixel)
        sim_loss = jnp.sum(jnp.where(valid, sim_per, 0.0), keepdims=True)

        gram = lax.dot_general(centers, centers, (((1,), (1,)), ((), ())),
                               preferred_element_type=jnp.float32)
        ids_r = lax.broadcasted_iota(jnp.int32, (K, K), 0)
        ids_c = lax.broadcasted_iota(jnp.int32, (K, K), 1)
        offdiag = jnp.where(ids_r == ids_c, 0.0, jnp.maximum(gram, 0.0))
        colsum = lax.dot_general(offdiag, rn, (((1,), (0,)), ((), ())),
                                 preferred_element_type=jnp.float32)
        diag_cos = (cn2 - _EPS) / cn2
        per_row = (colsum * rn + (1.0 - diag_cos)) / K
        diff_loss = jnp.sum(jnp.where(valid, per_row, 0.0), keepdims=True)

        out_ref[...] = sim_loss + diff_loss


def _embedding_loss(inputs_nchw, targets, num_classes, *, tile_rows=64):
    N, C, H, W = inputs_nchw.shape
    K = num_classes
    t = targets.astype(jnp.int32)

    R = tile_rows
    while R > 8 and H % R != 0:
        R //= 2
    if H % R != 0:
        R = H
    n_tiles = H // R
    n_steps = N * n_tiles

    loss = pl.pallas_call(
        functools.partial(_stats_loss_kernel, num_classes=K, n_steps=n_steps),
        out_shape=jax.ShapeDtypeStruct((1, 1), jnp.float32),
        grid_spec=pltpu.PrefetchScalarGridSpec(
            num_scalar_prefetch=0, grid=(n_steps,),
            in_specs=[pl.BlockSpec((None, C, R, W),
                                   lambda q: (q // n_tiles, 0,
                                              q % n_tiles, 0)),
                      pl.BlockSpec((None, R, W),
                                   lambda q: (q // n_tiles, q % n_tiles, 0))],
            out_specs=pl.BlockSpec((1, 1), lambda q: (0, 0)),
            scratch_shapes=[pltpu.VMEM((K, C), jnp.float32),
                            pltpu.VMEM((K, C), jnp.float32),
                            pltpu.VMEM((K, 1), jnp.float32)]),
        compiler_params=pltpu.CompilerParams(
            dimension_semantics=("arbitrary",)),
    )(inputs_nchw, t)

    return loss.reshape(1)


def kernel(inputs_nchw, targets):
    return _embedding_loss(inputs_nchw, targets, _K)
